# baseline (speedup 1.0000x reference)
import dataclasses
import functools

import jax
import jax.numpy as jnp
from jax import lax
from jax.experimental import pallas as pl
from jax.experimental.pallas import tpu as pltpu
from jax.experimental.pallas import tpu_sc as plsc

_VOCAB = 113
_VPAD = 128
_D = 512
_L = 200
_NC = 2
_NS = 16
_NW = _NC * _NS
_LANES = 16
_W = 32
_NBUF = 2


def _pos_encoding(max_length, d_model):
    even_i = jnp.arange(0, d_model, 2).astype(jnp.float32)
    denominator = jnp.power(jnp.float32(10000.0), even_i / d_model)
    position = jnp.arange(max_length, dtype=jnp.float32).reshape(max_length, 1)
    even_pe = jnp.sin(position / denominator)
    odd_pe = jnp.cos(position / denominator)
    return jnp.stack([even_pe, odd_pe], axis=2).reshape(max_length, d_model)


def _pack_pairs(arr_f32):
    bf = arr_f32.astype(jnp.bfloat16)
    r = bf.reshape(-1, _D // 32, 2, 16).transpose(0, 1, 3, 2)
    return jax.lax.bitcast_convert_type(r, jnp.int32).reshape(-1)


def _make_sc_lookup(n_tokens):
    per_w = n_tokens // _NW
    n_chunks = per_w // _W
    n_groups = n_chunks // _NBUF
    ch = _W * _D
    mesh = plsc.VectorSubcoreMesh(core_axis_name="c", subcore_axis_name="s")
    cp = pltpu.CompilerParams()
    if "needs_layout_passes" in pltpu.CompilerParams.__dataclass_fields__:
        cp = dataclasses.replace(cp, needs_layout_passes=False)

    @functools.partial(
        pl.kernel, mesh=mesh, compiler_params=cp,
        out_type=jax.ShapeDtypeStruct((n_tokens * _D,), jnp.float32),
        scratch_types=[
            pltpu.VMEM((_VPAD * _D // 2,), jnp.int32),
            pltpu.VMEM((_L * _D // 2,), jnp.int32),
            pltpu.VMEM((_NBUF * _W,), jnp.int32),
            pltpu.VMEM((_NBUF, ch), jnp.float32),
            pltpu.SemaphoreType.DMA((_NBUF,)),
        ],
    )
    def sc_lookup(tab_hbm, pe_hbm, x_hbm, out_hbm,
                  tab_v, pe_v, x_v, out_v, wsem):
        wid = lax.axis_index("s") * _NC + lax.axis_index("c")
        base = wid * per_w
        pltpu.sync_copy(tab_hbm, tab_v)
        pltpu.sync_copy(pe_hbm, pe_v)

        def do_chunk(c, b):
            tok0 = base + c * _W
            pltpu.sync_copy(x_hbm.at[pl.ds(tok0, _W)],
                            x_v.at[pl.ds(b * _W, _W)])

            @pl.loop(0, _W // _LANES)
            def _(m):
                vtok = x_v[pl.ds(b * _W + m * _LANES, _LANES)]
                for ln in range(_LANES):
                    n = m * _LANES + ln
                    xn = vtok[ln]
                    tn = lax.rem(tok0 + n, _L)
                    ebase = xn * (_D // 2)
                    pbase = tn * (_D // 2)
                    obase = n * _D
                    mask = jnp.int32(-65536)
                    for g in range(_D // 32):
                        ei = tab_v[pl.ds(ebase + g * 16, _LANES)]
                        pi = pe_v[pl.ds(pbase + g * 16, _LANES)]
                        lo = (plsc.bitcast(ei << 16, jnp.float32)
                              + plsc.bitcast(pi << 16, jnp.float32))
                        hi = (plsc.bitcast(ei & mask, jnp.float32)
                              + plsc.bitcast(pi & mask, jnp.float32))
                        out_v[b, pl.ds(obase + g * 32, _LANES)] = lo
                        out_v[b, pl.ds(obase + g * 32 + 16, _LANES)] = hi

            pltpu.async_copy(out_v.at[b], out_hbm.at[pl.ds(tok0 * _D, ch)],
                             wsem.at[b])

        for b in range(_NBUF):
            do_chunk(b, b)

        @pl.loop(1, n_groups)
        def _(gi):
            for b in range(_NBUF):
                pltpu.make_async_copy(out_hbm.at[pl.ds(0, ch)], out_v.at[b],
                                      wsem.at[b]).wait()
                do_chunk(gi * _NBUF + b, b)

        for b in range(_NBUF):
            pltpu.make_async_copy(out_hbm.at[pl.ds(0, ch)], out_v.at[b],
                                  wsem.at[b]).wait()

    return sc_lookup


@jax.jit
def _run(x_flat, tab_packed, pe_packed):
    return _make_sc_lookup(x_flat.shape[0])(tab_packed, pe_packed, x_flat)


def kernel(x, table):
    batch, length = x.shape
    pe = _pos_encoding(_L, _D)
    table_pad = jnp.zeros((_VPAD, _D), jnp.float32).at[:_VOCAB].set(table)
    x_flat = x.astype(jnp.int32).reshape(batch * length)
    out = _run(x_flat, _pack_pairs(table_pad), _pack_pairs(pe))
    return out.reshape(batch, length, _D)


# speedup vs baseline: 6.2340x; 6.2340x over previous
import functools

import jax
import jax.numpy as jnp
from jax import lax
from jax.experimental import pallas as pl
from jax.experimental.pallas import tpu as pltpu
from jax.experimental.pallas import tpu_sc as plsc

_VOCAB = 113
_VPAD = 128
_D = 512
_L = 200
_NC = 2
_NS = 16
_NW = _NC * _NS
_LANES = 16
_W = 32
_NBUF = 4


def _pos_encoding(max_length, d_model):
    even_i = jnp.arange(0, d_model, 2).astype(jnp.float32)
    denominator = jnp.power(jnp.float32(10000.0), even_i / d_model)
    position = jnp.arange(max_length, dtype=jnp.float32).reshape(max_length, 1)
    even_pe = jnp.sin(position / denominator)
    odd_pe = jnp.cos(position / denominator)
    return jnp.stack([even_pe, odd_pe], axis=2).reshape(max_length, d_model)


def _ctable_body(table_ref, pe_ref, out_ref):
    out_ref[...] = table_ref[...][None, :, :] + pe_ref[...][:, None, :]


@jax.jit
def _build_ctable(table_pad, pe):
    t_blk = 8
    return pl.pallas_call(
        _ctable_body,
        grid=(_L // t_blk,),
        in_specs=[
            pl.BlockSpec((_VPAD, _D), lambda i: (0, 0)),
            pl.BlockSpec((t_blk, _D), lambda i: (i, 0)),
        ],
        out_specs=pl.BlockSpec((t_blk, _VPAD, _D), lambda i: (i, 0, 0)),
        out_shape=jax.ShapeDtypeStruct((_L, _VPAD, _D), jnp.float32),
    )(table_pad, pe)


def _make_sc_gather(n_tokens):
    per_w = n_tokens // _NW
    n_chunks = per_w // _W
    n_groups = n_chunks // _NBUF
    mesh = plsc.VectorSubcoreMesh(core_axis_name="c", subcore_axis_name="s")

    @functools.partial(
        pl.kernel, mesh=mesh,
        out_type=jax.ShapeDtypeStruct((n_tokens, _D), jnp.float32),
        scratch_types=[
            pltpu.VMEM((_NBUF, _W), jnp.int32),
            pltpu.VMEM((_NBUF, _W, _D), jnp.float32),
            pltpu.SemaphoreType.DMA((_NBUF,)),
            pltpu.SemaphoreType.DMA((_NBUF,)),
        ],
    )
    def sc_gather(ctable_hbm, x_hbm, out_hbm, idx_v, rows_v, gsem, wsem):
        wid = lax.axis_index("s") * _NC + lax.axis_index("c")
        base = wid * per_w

        def load_and_gather(c, b):
            off = base + c * _W
            pltpu.sync_copy(x_hbm.at[pl.ds(off, _W)], idx_v.at[b])

            @pl.loop(0, _W // _LANES)
            def _(k):
                lane_n = off + k * _LANES + lax.broadcasted_iota(
                    jnp.int32, (_LANES,), 0)
                t = lax.rem(lane_n, _L)
                sl = pl.ds(k * _LANES, _LANES)
                idx_v[b, sl] = idx_v[b, sl] + t * _VPAD

            pltpu.async_copy(ctable_hbm.at[idx_v.at[b]], rows_v.at[b],
                             gsem.at[b])

        for b in range(_NBUF):
            load_and_gather(b, b)

        @pl.loop(0, n_groups)
        def _(g):
            writes = []
            for b in range(_NBUF):
                c = g * _NBUF + b
                off = base + c * _W
                pltpu.make_async_copy(out_hbm.at[pl.ds(off, _W)],
                                      rows_v.at[b], gsem.at[b]).wait()
                writes.append(pltpu.async_copy(
                    rows_v.at[b], out_hbm.at[pl.ds(off, _W)], wsem.at[b]))
            for b in range(_NBUF):
                c2 = g * _NBUF + b + _NBUF
                writes[b].wait()

                @pl.when(c2 < n_chunks)
                def _():
                    load_and_gather(c2, b)

    return sc_gather


@jax.jit
def _run(x_flat, table_pad, pe):
    ctable = _build_ctable(table_pad, pe).reshape(_L * _VPAD, _D)
    return _make_sc_gather(x_flat.shape[0])(ctable, x_flat)


def kernel(x, table):
    batch, length = x.shape
    pe = _pos_encoding(_L, _D)
    table_pad = jnp.zeros((_VPAD, _D), jnp.float32).at[:_VOCAB].set(table)
    x_flat = x.astype(jnp.int32).reshape(batch * length)
    out = _run(x_flat, table_pad, pe)
    return out.reshape(batch, length, _D)
